# baseline (speedup 1.0000x reference)
import functools

import jax
import jax.numpy as jnp
from jax import lax
from jax.experimental import pallas as pl
from jax.experimental.pallas import tpu as pltpu
from jax.experimental.pallas import tpu_sc as plsc

B = 16384
VOCAB = 100000
EMB = 64
EPS = 1e-3
NC, NS = 2, 16
NW = NC * NS
BPW = B // NW
CHUNK = 128
NCHUNK = BPW // CHUNK
NROW = B // CHUNK
TCBLK = 2048
NBLK = B // TCBLK
RPB = TCBLK // CHUNK
NBIN = 96


TBLK = 2048
TGRID = (VOCAB + TBLK - 1) // TBLK


def _transpose_pad_kernel(embT_ref, out_ref):
    ident = (lax.broadcasted_iota(jnp.int32, (EMB, 128), 0)
             == lax.broadcasted_iota(jnp.int32, (EMB, 128), 1)
             ).astype(jnp.float32)
    out_ref[...] = lax.dot_general(
        embT_ref[...], ident, (((0,), (0,)), ((), ())),
        preferred_element_type=jnp.float32)


def _transpose_pad(embT):
    return pl.pallas_call(
        _transpose_pad_kernel,
        grid=(TGRID,),
        in_specs=[pl.BlockSpec((EMB, TBLK), lambda i: (0, i))],
        out_specs=pl.BlockSpec((TBLK, 128), lambda i: (i, 0)),
        out_shape=jax.ShapeDtypeStruct((VOCAB, 128), jnp.float32),
    )(embT)


def _sc_gather(emb128, code, idx3):
    mesh = plsc.VectorSubcoreMesh(core_axis_name="c", subcore_axis_name="s")

    @functools.partial(
        pl.kernel,
        mesh=mesh,
        out_type=(
            jax.ShapeDtypeStruct((B, 128), jnp.float32),
            jax.ShapeDtypeStruct((NROW, CHUNK), jnp.int32),
        ),
        scratch_types=[
            pltpu.VMEM((NCHUNK, CHUNK), jnp.int32),
            pltpu.VMEM((BPW, 128), jnp.float32),
            pltpu.VMEM((NCHUNK, CHUNK), jnp.int32),
            pltpu.SemaphoreType.DMA,
        ],
    )
    def k(emb_hbm, code_hbm, idx_hbm, x_out, cats_out, idx_v, rows_v, val_v, sem):
        wid = lax.axis_index("s") * NC + lax.axis_index("c")
        pltpu.sync_copy(idx_hbm.at[wid], idx_v)
        copies = []
        for j in range(NCHUNK):
            copies.append(pltpu.async_copy(
                emb_hbm.at[idx_v.at[j]], rows_v.at[pl.ds(j * CHUNK, CHUNK)], sem))
            copies.append(pltpu.async_copy(
                code_hbm.at[idx_v.at[j]], val_v.at[j], sem))
        for c in copies:
            c.wait()
        pltpu.sync_copy(rows_v, x_out.at[pl.ds(wid * BPW, BPW)])
        pltpu.sync_copy(val_v, cats_out.at[pl.ds(wid * NCHUNK, NCHUNK)])

    return k(emb128, code, idx3)


def _onehot_t(crows):
    u = lax.broadcasted_iota(jnp.int32, (NBIN, 1, 1), 0)
    shift = jnp.where(u < 32, 0, jnp.where(u < 64, 5, 10))
    binval = u % 32
    val = lax.shift_right_logical(crows[None, :, :], shift) & 31
    ohf = jnp.where(val == binval, jnp.float32(1), jnp.float32(0))
    return ohf.astype(jnp.bfloat16)


def _fdot(a, b):
    return jnp.dot(a, b, preferred_element_type=jnp.float32)


def _tdot(a, b):
    return lax.dot_general(a, b, (((0,), (0,)), ((), ())),
                           preferred_element_type=jnp.float32)


def _tc_kernel(x_ref, cats_ref, gt_ref, grt_ref, ct_ref,
               w1_ref, w2_ref, w3_ref, w4_ref,
               g1_ref, g2_ref, g3_ref, g4_ref,
               b1_ref, b2_ref, b3_ref, b4_ref,
               out_ref, gw_ref, bias_ref, w1b_ref,
               accs_ref, accq_ref, ohacc_ref):
    i = pl.program_id(0)
    binv = jnp.float32(1.0 / B)

    @pl.when(i == 0)
    def _():
        accs_ref[...] = jnp.zeros((1, 128), jnp.float32)
        accq_ref[...] = jnp.zeros((1, 128), jnp.float32)
        ohacc_ref[...] = jnp.zeros((NBIN, CHUNK), jnp.float32)

    @pl.when(i < NBLK)
    def _():
        onesb = jnp.ones((1, TCBLK), jnp.bfloat16)
        xc = x_ref[...].astype(jnp.bfloat16)
        accs_ref[...] += _fdot(onesb, xc)
        accq_ref[...] += _fdot(onesb, xc * xc)
        oh3 = _onehot_t(cats_ref[pl.ds(i * RPB, RPB), :])
        acc = ohacc_ref[...]
        for c in range(RPB):
            acc += oh3[:, c, :].astype(jnp.float32)
        ohacc_ref[...] = acc

    @pl.when(i == NBLK)
    def _():
        cnt_row = jnp.transpose(
            jnp.sum(ohacc_ref[...], axis=1, keepdims=True))
        mean1 = accs_ref[:, 0:EMB] * binv
        var1 = accq_ref[:, 0:EMB] * binv - mean1 * mean1
        s1 = g1_ref[...] * lax.rsqrt(var1 + EPS)
        w1 = w1_ref[...]
        w1b_ref[0:EMB, :] = (w1 * jnp.transpose(s1)).astype(jnp.bfloat16)
        w1b_ref[EMB:128, :] = jnp.zeros((128 - EMB, 128), jnp.bfloat16)
        bias = _fdot(b1_ref[...] - mean1 * s1, w1)
        for off, t_ref, w_ref, g_ref, b_ref in (
                (0, gt_ref, w2_ref, g2_ref, b2_ref),
                (32, grt_ref, w3_ref, g3_ref, b3_ref),
                (64, ct_ref, w4_ref, g4_ref, b4_ref)):
            t = t_ref[...]
            cnt = cnt_row[:, off:off + 32]
            mean = _fdot(cnt, t) * binv
            ex2 = _fdot(cnt, t * t) * binv
            var = ex2 - mean * mean
            s = g_ref[...] * lax.rsqrt(var + EPS)
            gw_ref[off:off + 32, :] = _fdot(t * s, w_ref[...]).astype(jnp.bfloat16)
            bias += _fdot(b_ref[...] - mean * s, w_ref[...])
        bias_ref[...] = bias

    @pl.when(i >= NBLK)
    def _():
        x = x_ref[...].astype(jnp.bfloat16)
        base = _fdot(x, w1b_ref[...]) + bias_ref[...]
        oh3 = _onehot_t(cats_ref[pl.ds((i - NBLK) * RPB, RPB), :])
        gw = gw_ref[...]
        for c in range(RPB):
            out_ref[c * CHUNK:(c + 1) * CHUNK, :] = (
                base[c * CHUNK:(c + 1) * CHUNK, :] + _tdot(oh3[:, c, :], gw))


def _tc_fuse(x, cats2, gt_p, grt_p, ct_p, w1, w2p, w3p, w4p,
             g1, g2p, g3p, g4p, b1, b2p, b3p, b4p):
    full = lambda shape: pl.BlockSpec(shape, lambda i: (0, 0))
    xspec = pl.BlockSpec((TCBLK, 128),
                         lambda i: (jnp.where(i < NBLK, i, i - NBLK), 0))
    return pl.pallas_call(
        _tc_kernel,
        grid=(2 * NBLK,),
        in_specs=[
            xspec, full((NROW, CHUNK)),
            full((32, 16)), full((32, 16)), full((32, 16)),
            full((EMB, 128)), full((16, 128)), full((16, 128)), full((16, 128)),
            full((1, EMB)), full((1, 16)), full((1, 16)), full((1, 16)),
            full((1, EMB)), full((1, 16)), full((1, 16)), full((1, 16)),
        ],
        out_specs=pl.BlockSpec(
            (TCBLK, 128), lambda i: (jnp.where(i < NBLK, 0, i - NBLK), 0)),
        out_shape=jax.ShapeDtypeStruct((B, 128), jnp.float32),
        scratch_shapes=[
            pltpu.VMEM((NBIN, 128), jnp.bfloat16),
            pltpu.VMEM((1, 128), jnp.float32),
            pltpu.VMEM((128, 128), jnp.bfloat16),
            pltpu.VMEM((1, 128), jnp.float32),
            pltpu.VMEM((1, 128), jnp.float32),
            pltpu.VMEM((NBIN, CHUNK), jnp.float32),
        ],
    )(x, cats2, gt_p, grt_p, ct_p, w1, w2p, w3p, w4p,
      g1, g2p, g3p, g4p, b1, b2p, b3p, b4p)


def kernel(article_id, group_map, graph_map, colour_map,
           emb_table, group_table, graph_table, colour_table,
           gamma, beta, W):
    code = group_map | (graph_map << 5) | (colour_map << 10)
    emb128 = _transpose_pad(emb_table.T)
    idx3 = article_id.reshape(NW, NCHUNK, CHUNK)

    gt_p = jnp.zeros((32, 16), jnp.float32).at[:20, :10].set(group_table)
    grt_p = jnp.zeros((32, 16), jnp.float32).at[:31, :15].set(graph_table)
    ct_p = jnp.zeros((32, 16), jnp.float32).at[:21, :10].set(colour_table)
    w1 = W[:64]
    w2p = jnp.zeros((16, 128), jnp.float32).at[:10].set(W[64:74])
    w3p = jnp.zeros((16, 128), jnp.float32).at[:15].set(W[74:89])
    w4p = jnp.zeros((16, 128), jnp.float32).at[:10].set(W[89:99])
    g1 = gamma[:64].reshape(1, EMB)
    b1 = beta[:64].reshape(1, EMB)
    g2p = jnp.ones((1, 16), jnp.float32).at[0, :10].set(gamma[64:74])
    g3p = jnp.ones((1, 16), jnp.float32).at[0, :15].set(gamma[74:89])
    g4p = jnp.ones((1, 16), jnp.float32).at[0, :10].set(gamma[89:99])
    b2p = jnp.zeros((1, 16), jnp.float32).at[0, :10].set(beta[64:74])
    b3p = jnp.zeros((1, 16), jnp.float32).at[0, :15].set(beta[74:89])
    b4p = jnp.zeros((1, 16), jnp.float32).at[0, :10].set(beta[89:99])

    x, cats2 = _sc_gather(emb128, code, idx3)

    return _tc_fuse(x, cats2, gt_p, grt_p, ct_p, w1, w2p, w3p, w4p,
                    g1, g2p, g3p, g4p, b1, b2p, b3p, b4p)


# speedup vs baseline: 1.2307x; 1.2307x over previous
import functools

import jax
import jax.numpy as jnp
from jax import lax
from jax.experimental import pallas as pl
from jax.experimental.pallas import tpu as pltpu
from jax.experimental.pallas import tpu_sc as plsc

B = 16384
VOCAB = 100000
EMB = 64
EPS = 1e-3
NC, NS = 2, 16
NW = NC * NS
BPW = B // NW
CHUNK = 128
NCHUNK = BPW // CHUNK
NROW = B // CHUNK
TCBLK = 2048
NBLK = B // TCBLK
RPB = TCBLK // CHUNK
NBIN = 96
TBLK = 4096
TGRID = (VOCAB + TBLK - 1) // TBLK


def _transpose_pad_kernel(embT_ref, out_ref):
    ident = (lax.broadcasted_iota(jnp.int32, (EMB, 128), 0)
             == lax.broadcasted_iota(jnp.int32, (EMB, 128), 1)
             ).astype(jnp.float32)
    out_ref[...] = lax.dot_general(
        embT_ref[...], ident, (((0,), (0,)), ((), ())),
        preferred_element_type=jnp.float32)


def _transpose_pad(embT):
    return pl.pallas_call(
        _transpose_pad_kernel,
        grid=(TGRID,),
        in_specs=[pl.BlockSpec((EMB, TBLK), lambda i: (0, i))],
        out_specs=pl.BlockSpec((TBLK, 128), lambda i: (i, 0)),
        out_shape=jax.ShapeDtypeStruct((VOCAB, 128), jnp.float32),
    )(embT)


def _sc_gather(emb128, code, idx3):
    mesh = plsc.VectorSubcoreMesh(core_axis_name="c", subcore_axis_name="s")

    @functools.partial(
        pl.kernel,
        mesh=mesh,
        out_type=(
            jax.ShapeDtypeStruct((B, 128), jnp.float32),
            jax.ShapeDtypeStruct((NROW, CHUNK), jnp.int32),
        ),
        scratch_types=[
            pltpu.VMEM((NCHUNK, CHUNK), jnp.int32),
            pltpu.VMEM((BPW, 128), jnp.float32),
            pltpu.VMEM((NCHUNK, CHUNK), jnp.int32),
            pltpu.SemaphoreType.DMA,
        ],
    )
    def k(emb_hbm, code_hbm, idx_hbm, x_out, cats_out, idx_v, rows_v, val_v, sem):
        wid = lax.axis_index("s") * NC + lax.axis_index("c")
        pltpu.sync_copy(idx_hbm.at[wid], idx_v)
        copies = []
        for j in range(NCHUNK):
            copies.append(pltpu.async_copy(
                emb_hbm.at[idx_v.at[j]], rows_v.at[pl.ds(j * CHUNK, CHUNK)], sem))
            copies.append(pltpu.async_copy(
                code_hbm.at[idx_v.at[j]], val_v.at[j], sem))
        for c in copies:
            c.wait()
        pltpu.sync_copy(rows_v, x_out.at[pl.ds(wid * BPW, BPW)])
        pltpu.sync_copy(val_v, cats_out.at[pl.ds(wid * NCHUNK, NCHUNK)])

    return k(emb128, code, idx3)


def _onehot_t(crows):
    u = lax.broadcasted_iota(jnp.int32, (NBIN, 1, 1), 0)
    shift = jnp.where(u < 32, 0, jnp.where(u < 64, 5, 10))
    binval = u % 32
    val = lax.shift_right_logical(crows[None, :, :], shift) & 31
    ohf = jnp.where(val == binval, jnp.float32(1), jnp.float32(0))
    return ohf.astype(jnp.bfloat16)


def _fdot(a, b):
    return jnp.dot(a, b, preferred_element_type=jnp.float32)


def _tdot(a, b):
    return lax.dot_general(a, b, (((0,), (0,)), ((), ())),
                           preferred_element_type=jnp.float32)


def _tc_kernel(x_ref, cats_ref, gtT_ref, grtT_ref, ctT_ref, w_ref,
               gamma_ref, beta_ref,
               out_ref, gw_ref, bias_ref, w1b_ref):
    i = pl.program_id(0)
    binv = jnp.float32(1.0 / B)

    @pl.when(i == 0)
    def _():
        onesb = jnp.ones((1, TCBLK), jnp.bfloat16)
        acc_s = jnp.zeros((1, 128), jnp.float32)
        acc_q = jnp.zeros((1, 128), jnp.float32)
        ohacc = jnp.zeros((NBIN, CHUNK), jnp.float32)
        for k in range(NBLK):
            xc = x_ref[k * TCBLK:(k + 1) * TCBLK, :].astype(jnp.bfloat16)
            acc_s += _fdot(onesb, xc)
            acc_q += _fdot(onesb, xc * xc)
            oh3 = _onehot_t(cats_ref[k * RPB:(k + 1) * RPB, :])
            for c in range(RPB):
                ohacc += oh3[:, c, :].astype(jnp.float32)
        cnt_col = jnp.sum(ohacc, axis=1, keepdims=True)
        g1 = gamma_ref[0:EMB].reshape(1, EMB)
        b1 = beta_ref[0:EMB].reshape(1, EMB)
        mean1 = acc_s[:, 0:EMB] * binv
        var1 = acc_q[:, 0:EMB] * binv - mean1 * mean1
        s1 = g1 * lax.rsqrt(var1 + EPS)
        w1 = w_ref[0:EMB, :]
        w1b_ref[0:EMB, :] = (w1 * jnp.transpose(s1)).astype(jnp.bfloat16)
        w1b_ref[EMB:128, :] = jnp.zeros((128 - EMB, 128), jnp.bfloat16)
        gw_ref[...] = jnp.zeros((NBIN, 128), jnp.bfloat16)
        bias = _fdot(b1 - mean1 * s1, w1)
        for off, tT_ref, nc, wlo, whi in (
                (0, gtT_ref, 20, 64, 74),
                (32, grtT_ref, 31, 74, 89),
                (64, ctT_ref, 21, 89, 99)):
            tT = tT_ref[...]
            wp = w_ref[wlo:whi, :]
            nf = whi - wlo
            g = jnp.transpose(gamma_ref[wlo:whi].reshape(1, nf))
            b = jnp.transpose(beta_ref[wlo:whi].reshape(1, nf))
            cnt = cnt_col[off:off + nc, :]
            mean = _fdot(tT, cnt) * binv
            ex2 = _fdot(tT * tT, cnt) * binv
            var = ex2 - mean * mean
            s = g * lax.rsqrt(var + EPS)
            gw_ref[off:off + nc, :] = _tdot(tT * s, wp).astype(jnp.bfloat16)
            bias += _tdot(b - mean * s, wp)
        bias_ref[...] = bias

    x = x_ref[pl.ds(i * TCBLK, TCBLK), :].astype(jnp.bfloat16)
    base = _fdot(x, w1b_ref[...]) + bias_ref[...]
    oh3 = _onehot_t(cats_ref[pl.ds(i * RPB, RPB), :])
    gw = gw_ref[...]
    for c in range(RPB):
        out_ref[c * CHUNK:(c + 1) * CHUNK, :] = (
            base[c * CHUNK:(c + 1) * CHUNK, :] + _tdot(oh3[:, c, :], gw))


def _tc_fuse(x, cats2, gtT, grtT, ctT, W, gamma, beta):
    full = lambda shape: pl.BlockSpec(shape, lambda i: tuple(0 for _ in shape))
    return pl.pallas_call(
        _tc_kernel,
        grid=(NBLK,),
        in_specs=[full((B, 128)), full((NROW, CHUNK)),
                  full((10, 20)), full((15, 31)), full((10, 21)),
                  full((99, 128)), full((99,)), full((99,))],
        out_specs=pl.BlockSpec((TCBLK, 128), lambda i: (i, 0)),
        out_shape=jax.ShapeDtypeStruct((B, 128), jnp.float32),
        scratch_shapes=[
            pltpu.VMEM((NBIN, 128), jnp.bfloat16),
            pltpu.VMEM((1, 128), jnp.float32),
            pltpu.VMEM((128, 128), jnp.bfloat16),
        ],
    )(x, cats2, gtT, grtT, ctT, W, gamma, beta)


def kernel(article_id, group_map, graph_map, colour_map,
           emb_table, group_table, graph_table, colour_table,
           gamma, beta, W):
    code = group_map | (graph_map << 5) | (colour_map << 10)
    emb128 = _transpose_pad(emb_table.T)
    idx3 = article_id.reshape(NW, NCHUNK, CHUNK)

    x, cats2 = _sc_gather(emb128, code, idx3)

    return _tc_fuse(x, cats2, group_table.T, graph_table.T, colour_table.T,
                    W, gamma, beta)


# speedup vs baseline: 1.3918x; 1.1309x over previous
import functools

import jax
import jax.numpy as jnp
from jax import lax
from jax.experimental import pallas as pl
from jax.experimental.pallas import tpu as pltpu
from jax.experimental.pallas import tpu_sc as plsc

B = 16384
VOCAB = 100000
EMB = 64
EPS = 1e-3
NC, NS = 2, 16
NW = NC * NS
BPW = B // NW
CHUNK = 128
NCHUNK = BPW // CHUNK
NROW = B // CHUNK
TCBLK = 2048
NBLK = B // TCBLK
RPB = TCBLK // CHUNK
NBIN = 96
TBLK = 8192
TGRID = (VOCAB + TBLK - 1) // TBLK


def _transpose_pad_kernel(embT_ref, gm_ref, grm_ref, cm_ref, out_ref, code_ref):
    ident = (lax.broadcasted_iota(jnp.int32, (EMB, 128), 0)
             == lax.broadcasted_iota(jnp.int32, (EMB, 128), 1)
             ).astype(jnp.float32)
    out_ref[...] = lax.dot_general(
        embT_ref[...], ident, (((0,), (0,)), ((), ())),
        preferred_element_type=jnp.float32)
    code_ref[...] = (gm_ref[...] | (grm_ref[...] << 5) | (cm_ref[...] << 10))


def _transpose_pad(embT, group_map, graph_map, colour_map):
    return pl.pallas_call(
        _transpose_pad_kernel,
        grid=(TGRID,),
        in_specs=[pl.BlockSpec((EMB, TBLK), lambda i: (0, i)),
                  pl.BlockSpec((TBLK,), lambda i: (i,)),
                  pl.BlockSpec((TBLK,), lambda i: (i,)),
                  pl.BlockSpec((TBLK,), lambda i: (i,))],
        out_specs=[pl.BlockSpec((TBLK, 128), lambda i: (i, 0)),
                   pl.BlockSpec((TBLK,), lambda i: (i,))],
        out_shape=[jax.ShapeDtypeStruct((VOCAB, 128), jnp.float32),
                   jax.ShapeDtypeStruct((VOCAB,), jnp.int32)],
    )(embT, group_map, graph_map, colour_map)


def _sc_gather(emb128, code, idx3):
    mesh = plsc.VectorSubcoreMesh(core_axis_name="c", subcore_axis_name="s")

    @functools.partial(
        pl.kernel,
        mesh=mesh,
        out_type=(
            jax.ShapeDtypeStruct((B, 128), jnp.float32),
            jax.ShapeDtypeStruct((NROW, CHUNK), jnp.int32),
        ),
        scratch_types=[
            pltpu.VMEM((NCHUNK, CHUNK), jnp.int32),
            pltpu.VMEM((BPW, 128), jnp.float32),
            pltpu.VMEM((NCHUNK, CHUNK), jnp.int32),
            pltpu.SemaphoreType.DMA,
        ],
    )
    def k(emb_hbm, code_hbm, idx_hbm, x_out, cats_out, idx_v, rows_v, val_v, sem):
        wid = lax.axis_index("s") * NC + lax.axis_index("c")
        pltpu.sync_copy(idx_hbm.at[wid], idx_v)
        copies = []
        for j in range(NCHUNK):
            copies.append(pltpu.async_copy(
                emb_hbm.at[idx_v.at[j]], rows_v.at[pl.ds(j * CHUNK, CHUNK)], sem))
            copies.append(pltpu.async_copy(
                code_hbm.at[idx_v.at[j]], val_v.at[j], sem))
        for c in copies:
            c.wait()
        pltpu.sync_copy(rows_v, x_out.at[pl.ds(wid * BPW, BPW)])
        pltpu.sync_copy(val_v, cats_out.at[pl.ds(wid * NCHUNK, NCHUNK)])

    return k(emb128, code, idx3)


def _onehot_t(crows):
    u = lax.broadcasted_iota(jnp.int32, (NBIN, 1, 1), 0)
    shift = jnp.where(u < 32, 0, jnp.where(u < 64, 5, 10))
    binval = u % 32
    val = lax.shift_right_logical(crows[None, :, :], shift) & 31
    ohf = jnp.where(val == binval, jnp.float32(1), jnp.float32(0))
    return ohf.astype(jnp.bfloat16)


def _fdot(a, b):
    return jnp.dot(a, b, preferred_element_type=jnp.float32)


def _tdot(a, b):
    return lax.dot_general(a, b, (((0,), (0,)), ((), ())),
                           preferred_element_type=jnp.float32)


def _tc_kernel(x_ref, cats_ref, gtT_ref, grtT_ref, ctT_ref, w_ref,
               gamma_ref, beta_ref,
               out_ref, gw_ref, bias_ref, w1b_ref):
    i = pl.program_id(0)
    binv = jnp.float32(1.0 / B)

    @pl.when(i == 0)
    def _():
        onesb = jnp.ones((1, TCBLK), jnp.bfloat16)
        acc_s = jnp.zeros((1, 128), jnp.float32)
        acc_q = jnp.zeros((1, 128), jnp.float32)
        ohacc = jnp.zeros((NBIN, CHUNK), jnp.float32)
        for k in range(NBLK):
            xc = x_ref[k * TCBLK:(k + 1) * TCBLK, :].astype(jnp.bfloat16)
            acc_s += _fdot(onesb, xc)
            acc_q += _fdot(onesb, xc * xc)
            oh3 = _onehot_t(cats_ref[k * RPB:(k + 1) * RPB, :])
            for c in range(RPB):
                ohacc += oh3[:, c, :].astype(jnp.float32)
        cnt_col = jnp.sum(ohacc, axis=1, keepdims=True)
        g1 = gamma_ref[0:EMB].reshape(1, EMB)
        b1 = beta_ref[0:EMB].reshape(1, EMB)
        mean1 = acc_s[:, 0:EMB] * binv
        var1 = acc_q[:, 0:EMB] * binv - mean1 * mean1
        s1 = g1 * lax.rsqrt(var1 + EPS)
        w1 = w_ref[0:EMB, :]
        w1b_ref[0:EMB, :] = (w1 * jnp.transpose(s1)).astype(jnp.bfloat16)
        w1b_ref[EMB:128, :] = jnp.zeros((128 - EMB, 128), jnp.bfloat16)
        gw_ref[...] = jnp.zeros((NBIN, 128), jnp.bfloat16)
        bias = _fdot(b1 - mean1 * s1, w1)
        for off, tT_ref, nc, wlo, whi in (
                (0, gtT_ref, 20, 64, 74),
                (32, grtT_ref, 31, 74, 89),
                (64, ctT_ref, 21, 89, 99)):
            tT = tT_ref[...]
            wp = w_ref[wlo:whi, :]
            nf = whi - wlo
            g = jnp.transpose(gamma_ref[wlo:whi].reshape(1, nf))
            b = jnp.transpose(beta_ref[wlo:whi].reshape(1, nf))
            cnt = cnt_col[off:off + nc, :]
            mean = _fdot(tT, cnt) * binv
            ex2 = _fdot(tT * tT, cnt) * binv
            var = ex2 - mean * mean
            s = g * lax.rsqrt(var + EPS)
            gw_ref[off:off + nc, :] = _tdot(tT * s, wp).astype(jnp.bfloat16)
            bias += _tdot(b - mean * s, wp)
        bias_ref[...] = bias

    x = x_ref[pl.ds(i * TCBLK, TCBLK), :].astype(jnp.bfloat16)
    base = _fdot(x, w1b_ref[...]) + bias_ref[...]
    oh3 = _onehot_t(cats_ref[pl.ds(i * RPB, RPB), :])
    gw = gw_ref[...]
    for c in range(RPB):
        out_ref[c * CHUNK:(c + 1) * CHUNK, :] = (
            base[c * CHUNK:(c + 1) * CHUNK, :] + _tdot(oh3[:, c, :], gw))


def _tc_fuse(x, cats2, gtT, grtT, ctT, W, gamma, beta):
    full = lambda shape: pl.BlockSpec(shape, lambda i: tuple(0 for _ in shape))
    return pl.pallas_call(
        _tc_kernel,
        grid=(NBLK,),
        in_specs=[full((B, 128)), full((NROW, CHUNK)),
                  full((10, 20)), full((15, 31)), full((10, 21)),
                  full((99, 128)), full((99,)), full((99,))],
        out_specs=pl.BlockSpec((TCBLK, 128), lambda i: (i, 0)),
        out_shape=jax.ShapeDtypeStruct((B, 128), jnp.float32),
        scratch_shapes=[
            pltpu.VMEM((NBIN, 128), jnp.bfloat16),
            pltpu.VMEM((1, 128), jnp.float32),
            pltpu.VMEM((128, 128), jnp.bfloat16),
        ],
    )(x, cats2, gtT, grtT, ctT, W, gamma, beta)


def kernel(article_id, group_map, graph_map, colour_map,
           emb_table, group_table, graph_table, colour_table,
           gamma, beta, W):
    emb128, code = _transpose_pad(emb_table.T, group_map, graph_map, colour_map)
    idx3 = article_id.reshape(NW, NCHUNK, CHUNK)

    x, cats2 = _sc_gather(emb128, code, idx3)

    return _tc_fuse(x, cats2, group_table.T, graph_table.T, colour_table.T,
                    W, gamma, beta)


# speedup vs baseline: 1.4309x; 1.0281x over previous
import functools

import jax
import jax.numpy as jnp
from jax import lax
from jax.experimental import pallas as pl
from jax.experimental.pallas import tpu as pltpu
from jax.experimental.pallas import tpu_sc as plsc

B = 16384
VOCAB = 100000
EMB = 64
EPS = 1e-3
NC, NS = 2, 16
NW = NC * NS
BPW = B // NW
CHUNK = 128
NCHUNK = BPW // CHUNK
NROW = B // CHUNK
TCBLK = 2048
NBLK = B // TCBLK
RPB = TCBLK // CHUNK
NBIN = 96
TBLK = 16384
TGRID = (VOCAB + TBLK - 1) // TBLK


def _transpose_pad_kernel(embT_ref, gm_ref, grm_ref, cm_ref, out_ref, code_ref):
    ident = (lax.broadcasted_iota(jnp.int32, (EMB, 128), 0)
             == lax.broadcasted_iota(jnp.int32, (EMB, 128), 1)
             ).astype(jnp.float32)
    out_ref[...] = lax.dot_general(
        embT_ref[...], ident, (((0,), (0,)), ((), ())),
        preferred_element_type=jnp.float32)
    code_ref[...] = (gm_ref[...] | (grm_ref[...] << 5) | (cm_ref[...] << 10))


def _transpose_pad(embT, group_map, graph_map, colour_map):
    return pl.pallas_call(
        _transpose_pad_kernel,
        grid=(TGRID,),
        in_specs=[pl.BlockSpec((EMB, TBLK), lambda i: (0, i)),
                  pl.BlockSpec((TBLK,), lambda i: (i,)),
                  pl.BlockSpec((TBLK,), lambda i: (i,)),
                  pl.BlockSpec((TBLK,), lambda i: (i,))],
        out_specs=[pl.BlockSpec((TBLK, 128), lambda i: (i, 0)),
                   pl.BlockSpec((TBLK,), lambda i: (i,))],
        out_shape=[jax.ShapeDtypeStruct((VOCAB, 128), jnp.float32),
                   jax.ShapeDtypeStruct((VOCAB,), jnp.int32)],
    )(embT, group_map, graph_map, colour_map)


def _sc_gather(emb128, code, idx3):
    mesh = plsc.VectorSubcoreMesh(core_axis_name="c", subcore_axis_name="s")

    @functools.partial(
        pl.kernel,
        mesh=mesh,
        out_type=(
            jax.ShapeDtypeStruct((B, 128), jnp.float32),
            jax.ShapeDtypeStruct((NROW, CHUNK), jnp.int32),
        ),
        scratch_types=[
            pltpu.VMEM((NCHUNK, CHUNK), jnp.int32),
            pltpu.VMEM((BPW, 128), jnp.float32),
            pltpu.VMEM((NCHUNK, CHUNK), jnp.int32),
            pltpu.SemaphoreType.DMA,
        ],
    )
    def k(emb_hbm, code_hbm, idx_hbm, x_out, cats_out, idx_v, rows_v, val_v, sem):
        wid = lax.axis_index("s") * NC + lax.axis_index("c")
        pltpu.sync_copy(idx_hbm.at[wid], idx_v)
        copies = []
        for j in range(NCHUNK):
            copies.append(pltpu.async_copy(
                emb_hbm.at[idx_v.at[j]], rows_v.at[pl.ds(j * CHUNK, CHUNK)], sem))
            copies.append(pltpu.async_copy(
                code_hbm.at[idx_v.at[j]], val_v.at[j], sem))
        for c in copies:
            c.wait()
        pltpu.sync_copy(rows_v, x_out.at[pl.ds(wid * BPW, BPW)])
        pltpu.sync_copy(val_v, cats_out.at[pl.ds(wid * NCHUNK, NCHUNK)])

    return k(emb128, code, idx3)


def _onehot_t(crows):
    u = lax.broadcasted_iota(jnp.int32, (NBIN, 1, 1), 0)
    shift = jnp.where(u < 32, 0, jnp.where(u < 64, 5, 10))
    binval = u % 32
    val = lax.shift_right_logical(crows[None, :, :], shift) & 31
    ohf = jnp.where(val == binval, jnp.float32(1), jnp.float32(0))
    return ohf.astype(jnp.bfloat16)


def _fdot(a, b):
    return jnp.dot(a, b, preferred_element_type=jnp.float32)


def _tdot(a, b):
    return lax.dot_general(a, b, (((0,), (0,)), ((), ())),
                           preferred_element_type=jnp.float32)


def _tc_kernel(x_ref, cats_ref, gtT_ref, grtT_ref, ctT_ref, w_ref,
               gamma_ref, beta_ref,
               out_ref, gw_ref, bias_ref, w1b_ref):
    i = pl.program_id(0)
    binv = jnp.float32(1.0 / B)

    @pl.when(i == 0)
    def _():
        onesb = jnp.ones((1, TCBLK), jnp.bfloat16)
        acc_s = jnp.zeros((1, 128), jnp.float32)
        acc_q = jnp.zeros((1, 128), jnp.float32)
        ohacc = jnp.zeros((NBIN, CHUNK), jnp.float32)
        for k in range(NBLK):
            xc = x_ref[k * TCBLK:(k + 1) * TCBLK, :].astype(jnp.bfloat16)
            acc_s += _fdot(onesb, xc)
            acc_q += _fdot(onesb, xc * xc)
            oh3 = _onehot_t(cats_ref[k * RPB:(k + 1) * RPB, :])
            for c in range(RPB):
                ohacc += oh3[:, c, :].astype(jnp.float32)
        cnt_col = jnp.sum(ohacc, axis=1, keepdims=True)
        g1 = gamma_ref[0:EMB].reshape(1, EMB)
        b1 = beta_ref[0:EMB].reshape(1, EMB)
        mean1 = acc_s[:, 0:EMB] * binv
        var1 = acc_q[:, 0:EMB] * binv - mean1 * mean1
        s1 = g1 * lax.rsqrt(var1 + EPS)
        w1 = w_ref[0:EMB, :]
        w1b_ref[0:EMB, :] = (w1 * jnp.transpose(s1)).astype(jnp.bfloat16)
        w1b_ref[EMB:128, :] = jnp.zeros((128 - EMB, 128), jnp.bfloat16)
        gw_ref[...] = jnp.zeros((NBIN, 128), jnp.bfloat16)
        bias = _fdot(b1 - mean1 * s1, w1)
        for off, tT_ref, nc, wlo, whi in (
                (0, gtT_ref, 20, 64, 74),
                (32, grtT_ref, 31, 74, 89),
                (64, ctT_ref, 21, 89, 99)):
            tT = tT_ref[...]
            wp = w_ref[wlo:whi, :]
            nf = whi - wlo
            g = jnp.transpose(gamma_ref[wlo:whi].reshape(1, nf))
            b = jnp.transpose(beta_ref[wlo:whi].reshape(1, nf))
            cnt = cnt_col[off:off + nc, :]
            mean = _fdot(tT, cnt) * binv
            ex2 = _fdot(tT * tT, cnt) * binv
            var = ex2 - mean * mean
            s = g * lax.rsqrt(var + EPS)
            gw_ref[off:off + nc, :] = _tdot(tT * s, wp).astype(jnp.bfloat16)
            bias += _tdot(b - mean * s, wp)
        bias_ref[...] = bias

    x = x_ref[pl.ds(i * TCBLK, TCBLK), :].astype(jnp.bfloat16)
    base = _fdot(x, w1b_ref[...]) + bias_ref[...]
    oh3 = _onehot_t(cats_ref[pl.ds(i * RPB, RPB), :])
    gw = gw_ref[...]
    for c in range(RPB):
        out_ref[c * CHUNK:(c + 1) * CHUNK, :] = (
            base[c * CHUNK:(c + 1) * CHUNK, :] + _tdot(oh3[:, c, :], gw))


def _tc_fuse(x, cats2, gtT, grtT, ctT, W, gamma, beta):
    full = lambda shape: pl.BlockSpec(shape, lambda i: tuple(0 for _ in shape))
    return pl.pallas_call(
        _tc_kernel,
        grid=(NBLK,),
        in_specs=[full((B, 128)), full((NROW, CHUNK)),
                  full((10, 20)), full((15, 31)), full((10, 21)),
                  full((99, 128)), full((99,)), full((99,))],
        out_specs=pl.BlockSpec((TCBLK, 128), lambda i: (i, 0)),
        out_shape=jax.ShapeDtypeStruct((B, 128), jnp.float32),
        scratch_shapes=[
            pltpu.VMEM((NBIN, 128), jnp.bfloat16),
            pltpu.VMEM((1, 128), jnp.float32),
            pltpu.VMEM((128, 128), jnp.bfloat16),
        ],
    )(x, cats2, gtT, grtT, ctT, W, gamma, beta)


def kernel(article_id, group_map, graph_map, colour_map,
           emb_table, group_table, graph_table, colour_table,
           gamma, beta, W):
    emb128, code = _transpose_pad(emb_table.T, group_map, graph_map, colour_map)
    idx3 = article_id.reshape(NW, NCHUNK, CHUNK)

    x, cats2 = _sc_gather(emb128, code, idx3)

    return _tc_fuse(x, cats2, group_table.T, graph_table.T, colour_table.T,
                    W, gamma, beta)
